# baseline (speedup 1.0000x reference)
import functools

import jax
import jax.numpy as jnp
from jax import lax
from jax.experimental import pallas as pl
from jax.experimental.pallas import tpu as pltpu
from jax.experimental.pallas import tpu_sc as plsc

B = 1024
T = 50
VOCAB = 1000
NW = 32
NV8 = VOCAB // 8
NBT = B // 128


def _body(idx_hbm, table_hbm, out_hbm, idx_v, slab_v, buf_v, sem0, sem1):
    wid = lax.axis_index("s") * 2 + lax.axis_index("c")
    cnt = jnp.where(wid < 29, 4, 3)
    start = 4 * wid - jnp.maximum(wid - 29, 0)
    pltpu.sync_copy(idx_hbm, idx_v)

    def do_slab(k, carry):
        v8 = start + k
        pltpu.sync_copy(table_hbm.at[pl.ds(v8 * 8, 8)], slab_v)

        def do_pair(p, carry2):
            for par, sem in ((0, sem0), (1, sem1)):
                t = 2 * p + par

                @pl.when(p >= 1)
                def _wait():
                    pltpu.make_async_copy(
                        buf_v.at[par], out_hbm.at[0, 0], sem).wait()

                @functools.partial(plsc.parallel_loop, 0, NBT)
                def _bt_loop(bt):
                    for g in range(8):
                        idxv = idx_v[t, pl.ds(bt * 128 + g * 16, 16)]
                        for vv in range(8):
                            row = jnp.full((16,), vv, jnp.int32)
                            val = plsc.load_gather(slab_v, [row, idxv])
                            buf_v[par, bt, vv, pl.ds(g * 16, 16)] = val
                pltpu.async_copy(buf_v.at[par], out_hbm.at[t, v8], sem)
            return carry2

        lax.fori_loop(0, T // 2, do_pair, 0)
        pltpu.make_async_copy(buf_v.at[0], out_hbm.at[0, 0], sem0).wait()
        pltpu.make_async_copy(buf_v.at[1], out_hbm.at[0, 0], sem1).wait()
        return carry

    lax.fori_loop(0, cnt, do_slab, 0)


@jax.jit
def _gather(idx_t, table_t):
    mesh = plsc.VectorSubcoreMesh(core_axis_name="c", subcore_axis_name="s")
    f = functools.partial(
        pl.kernel,
        mesh=mesh,
        out_type=jax.ShapeDtypeStruct((T, NV8, NBT, 8, 128), jnp.float32),
        scratch_types=[
            pltpu.VMEM((T, B), jnp.int32),
            pltpu.VMEM((8, VOCAB), jnp.float32),
            pltpu.VMEM((2, NBT, 8, 128), jnp.float32),
            pltpu.SemaphoreType.DMA,
            pltpu.SemaphoreType.DMA,
        ],
        compiler_params=pltpu.CompilerParams(use_tc_tiling_on_sc=False, needs_layout_passes=False),
    )(_body)
    return f(idx_t, table_t)


def kernel(idx, table):
    out5 = _gather(idx.T, table.T)
    return out5.transpose(2, 4, 0, 1, 3).reshape(B, T, VOCAB)


# speedup vs baseline: 1.0009x; 1.0009x over previous
import functools

import jax
import jax.numpy as jnp
from jax import lax
from jax.experimental import pallas as pl
from jax.experimental.pallas import tpu as pltpu
from jax.experimental.pallas import tpu_sc as plsc

B = 1024
T = 50
VOCAB = 1000
NW = 32
NV8 = VOCAB // 8
NBT = B // 128


def _body(idx_hbm, table_hbm, out_hbm, idx_v, slab_v, buf_v, sem0, sem1):
    wid = lax.axis_index("s") * 2 + lax.axis_index("c")
    cnt = jnp.where(wid < 29, 4, 3)
    start = 4 * wid - jnp.maximum(wid - 29, 0)
    pltpu.sync_copy(idx_hbm, idx_v)

    def do_slab(k, carry):
        v8 = start + k
        pltpu.sync_copy(table_hbm.at[:, pl.ds(v8 * 8, 8)], slab_v)

        def do_pair(p, carry2):
            for par, sem in ((0, sem0), (1, sem1)):
                t = 2 * p + par

                @pl.when(p >= 1)
                def _wait():
                    pltpu.make_async_copy(
                        buf_v.at[par], out_hbm.at[0, 0], sem).wait()

                @functools.partial(plsc.parallel_loop, 0, NBT)
                def _bt_loop(bt):
                    for g in range(8):
                        idxv = idx_v[t, pl.ds(bt * 128 + g * 16, 16)]
                        for vv in range(8):
                            col = jnp.full((16,), vv, jnp.int32)
                            val = plsc.load_gather(slab_v, [idxv, col])
                            buf_v[par, bt, vv, pl.ds(g * 16, 16)] = val
                pltpu.async_copy(buf_v.at[par], out_hbm.at[t, v8], sem)
            return carry2

        lax.fori_loop(0, T // 2, do_pair, 0)
        pltpu.make_async_copy(buf_v.at[0], out_hbm.at[0, 0], sem0).wait()
        pltpu.make_async_copy(buf_v.at[1], out_hbm.at[0, 0], sem1).wait()
        return carry

    lax.fori_loop(0, cnt, do_slab, 0)


@jax.jit
def _gather(idx_t, table_t):
    mesh = plsc.VectorSubcoreMesh(core_axis_name="c", subcore_axis_name="s")
    f = functools.partial(
        pl.kernel,
        mesh=mesh,
        out_type=jax.ShapeDtypeStruct((T, NV8, NBT, 8, 128), jnp.float32),
        scratch_types=[
            pltpu.VMEM((T, B), jnp.int32),
            pltpu.VMEM((VOCAB, 8), jnp.float32),
            pltpu.VMEM((2, NBT, 8, 128), jnp.float32),
            pltpu.SemaphoreType.DMA,
            pltpu.SemaphoreType.DMA,
        ],
        compiler_params=pltpu.CompilerParams(use_tc_tiling_on_sc=False, needs_layout_passes=False),
    )(_body)
    return f(idx_t, table_t)


def kernel(idx, table):
    out5 = _gather(idx.T, table)
    return out5.transpose(2, 4, 0, 1, 3).reshape(B, T, VOCAB)


# speedup vs baseline: 1.0518x; 1.0508x over previous
import functools

import jax
import jax.numpy as jnp
from jax import lax
from jax.experimental import pallas as pl
from jax.experimental.pallas import tpu as pltpu
from jax.experimental.pallas import tpu_sc as plsc

B = 1024
T = 50
VOCAB = 1000
NW = 32
NV8 = VOCAB // 8
NBT = B // 128


def _body(idx_hbm, table_hbm, out_hbm, idx_v, slab_v, buf_v, sem0, sem1):
    wid = lax.axis_index("s") * 2 + lax.axis_index("c")
    has4 = wid < 29
    start = 4 * wid - jnp.maximum(wid - 29, 0)
    pltpu.sync_copy(idx_hbm, idx_v)

    @pl.when(has4)
    def _load4():
        pltpu.sync_copy(table_hbm.at[pl.ds(start * 8, 32)], slab_v)

    @pl.when(jnp.logical_not(has4))
    def _load3():
        pltpu.sync_copy(table_hbm.at[pl.ds(start * 8, 24)],
                        slab_v.at[pl.ds(0, 24)])

    def drain(par, sem):
        @pl.when(has4)
        def _d4():
            pltpu.make_async_copy(
                buf_v.at[par],
                out_hbm.at[0, pl.ds(0, 4), pl.ds(0, 4)], sem).wait()

        @pl.when(jnp.logical_not(has4))
        def _d3():
            pltpu.make_async_copy(
                buf_v.at[par, pl.ds(0, 3)],
                out_hbm.at[0, pl.ds(0, 3), pl.ds(0, 4)], sem).wait()

    def do_t(t, carry):
        for par, sem in ((0, sem0), (1, sem1)):
            h = par

            @pl.when(t >= 1)
            def _wait():
                drain(par, sem)

            @functools.partial(plsc.parallel_loop, 0, NBT // 2)
            def _bt_loop(lbt):
                for g in range(8):
                    idxv = idx_v[t, pl.ds((4 * h + lbt) * 128 + g * 16, 16)]
                    for k in range(4):
                        for vv in range(8):
                            row = jnp.full((16,), k * 8 + vv, jnp.int32)
                            val = plsc.load_gather(slab_v, [row, idxv])
                            buf_v[par, k, lbt, vv, pl.ds(g * 16, 16)] = val

            @pl.when(has4)
            def _w4():
                pltpu.async_copy(
                    buf_v.at[par],
                    out_hbm.at[t, pl.ds(start, 4), pl.ds(4 * h, 4)], sem)

            @pl.when(jnp.logical_not(has4))
            def _w3():
                pltpu.async_copy(
                    buf_v.at[par, pl.ds(0, 3)],
                    out_hbm.at[t, pl.ds(start, 3), pl.ds(4 * h, 4)], sem)
        return carry

    lax.fori_loop(0, T, do_t, 0)
    drain(0, sem0)
    drain(1, sem1)


@jax.jit
def _gather(idx_t, table_t):
    mesh = plsc.VectorSubcoreMesh(core_axis_name="c", subcore_axis_name="s")
    f = functools.partial(
        pl.kernel,
        mesh=mesh,
        out_type=jax.ShapeDtypeStruct((T, NV8, NBT, 8, 128), jnp.float32),
        scratch_types=[
            pltpu.VMEM((T, B), jnp.int32),
            pltpu.VMEM((32, VOCAB), jnp.float32),
            pltpu.VMEM((2, 4, NBT // 2, 8, 128), jnp.float32),
            pltpu.SemaphoreType.DMA,
            pltpu.SemaphoreType.DMA,
        ],
        compiler_params=pltpu.CompilerParams(
            use_tc_tiling_on_sc=False, needs_layout_passes=False),
    )(_body)
    return f(idx_t, table_t)


def kernel(idx, table):
    out5 = _gather(idx.T, table.T)
    return out5.transpose(2, 4, 0, 1, 3).reshape(B, T, VOCAB)
